# baseline (speedup 1.0000x reference)
import functools

import jax
import jax.numpy as jnp
from jax import lax
from jax.experimental import pallas as pl
from jax.experimental.pallas import tpu as pltpu
from jax.experimental.pallas import tpu_sc as plsc

NC = 2
NS = 16
NW = NC * NS
L = 16
WPAD = 312

BW = 8192
VS0 = 737280

TPW = 64
CHT = 16
CHW = CHT * 128
NFULL = 4
REMW = (TPW - NFULL * CHT) * 128
TAIL0 = VS0 + NW * TPW * 128
NTAILC = 5
VSCPAD = TAIL0 - VS0 + NTAILC * 128
DFULL = 296


def _matvec_tc(D):
    def mv(w_ref, x_ref, o1_ref, o2_ref):
        p = jax.lax.dot_general(
            w_ref[...], x_ref[...], (((1,), (0,)), ((), ())),
            preferred_element_type=jnp.float32)
        o1_ref[...] = p[0:1]
        o2_ref[...] = p[1:2]

    return pl.pallas_call(
        mv,
        grid=(VS0 // BW,),
        in_specs=[
            pl.BlockSpec((2, D), lambda i: (0, 0)),
            pl.BlockSpec((D, BW), lambda i: (0, i)),
        ],
        out_specs=[
            pl.BlockSpec((1, BW), lambda i: (0, i)),
            pl.BlockSpec((1, BW), lambda i: (0, i)),
        ],
        out_shape=[
            jax.ShapeDtypeStruct((1, VS0), jnp.float32),
            jax.ShapeDtypeStruct((1, VS0), jnp.float32),
        ],
    )


def _matvec_sc():
    DT = DFULL // 8

    mesh = plsc.VectorSubcoreMesh(core_axis_name="c", subcore_axis_name="s")

    @functools.partial(
        pl.kernel,
        out_type=(
            jax.ShapeDtypeStruct((VSCPAD,), jnp.float32),
            jax.ShapeDtypeStruct((VSCPAD,), jnp.float32),
        ),
        mesh=mesh,
        compiler_params=pltpu.CompilerParams(use_tc_tiling_on_sc=True),
        scratch_types=[
            pltpu.VMEM((8, CHW), jnp.float32),
            pltpu.VMEM((8, CHW), jnp.float32),
            pltpu.VMEM((8, CHW), jnp.float32),
            pltpu.VMEM((CHW,), jnp.float32),
            pltpu.VMEM((CHW,), jnp.float32),
            pltpu.VMEM((WPAD,), jnp.float32),
            pltpu.VMEM((WPAD,), jnp.float32),
            pltpu.VMEM((304, 128), jnp.float32),
            pltpu.SemaphoreType.DMA,
            pltpu.SemaphoreType.DMA,
            pltpu.SemaphoreType.DMA,
        ],
    )
    def k(tabT_hbm, taild_hbm, tail2_hbm, w1_hbm, w2_hbm, p1_hbm, p2_hbm,
          bufa, bufb, bufd, acc1, acc2, w1_v, w2_v, tbuf,
          sema, semb, semd):
        wid = lax.axis_index("s") * NC + lax.axis_index("c")
        pltpu.sync_copy(w1_hbm, w1_v)
        pltpu.sync_copy(w2_hbm, w2_v)

        def accumulate(strip, r, ng):
            wv1 = w1_v[pl.ds(8 * r, L)]
            wv2 = w2_v[pl.ds(8 * r, L)]

            def gbody(g, _):
                sl = pl.ds(g * L, L)
                a1 = acc1[sl]
                a2 = acc2[sl]
                for kk in range(8):
                    v = strip[kk, sl]
                    a1 = a1 + v * wv1[kk]
                    a2 = a2 + v * wv2[kk]
                acc1[sl] = a1
                acc2[sl] = a2
                return 0

            lax.fori_loop(0, ng, gbody, 0)

        def do_chunk(lv0, vw):
            ng = vw // L

            def zbody(g, _):
                z = jnp.zeros((L,), jnp.float32)
                acc1[pl.ds(g * L, L)] = z
                acc2[pl.ds(g * L, L)] = z
                return 0

            lax.fori_loop(0, ng, zbody, 0)

            cpd = pltpu.async_copy(
                taild_hbm.at[:, pl.ds(lv0, vw)],
                bufd.at[:, pl.ds(0, vw)], semd)
            pltpu.async_copy(
                tabT_hbm.at[pl.ds(0, 8), pl.ds(VS0 + lv0, vw)],
                bufa.at[:, pl.ds(0, vw)], sema)

            def rbody(r, _):
                @pl.when(r < DT - 1)
                def _():
                    nxt = r + 1

                    @pl.when(nxt % 2 == 0)
                    def _():
                        pltpu.async_copy(
                            tabT_hbm.at[pl.ds(8 * nxt, 8),
                                        pl.ds(VS0 + lv0, vw)],
                            bufa.at[:, pl.ds(0, vw)], sema)

                    @pl.when(nxt % 2 == 1)
                    def _():
                        pltpu.async_copy(
                            tabT_hbm.at[pl.ds(8 * nxt, 8),
                                        pl.ds(VS0 + lv0, vw)],
                            bufb.at[:, pl.ds(0, vw)], semb)

                @pl.when(r % 2 == 0)
                def _():
                    pltpu.make_async_copy(
                        tabT_hbm.at[pl.ds(0, 8), pl.ds(VS0 + lv0, vw)],
                        bufa.at[:, pl.ds(0, vw)], sema).wait()
                    accumulate(bufa, r, ng)

                @pl.when(r % 2 == 1)
                def _():
                    pltpu.make_async_copy(
                        tabT_hbm.at[pl.ds(0, 8), pl.ds(VS0 + lv0, vw)],
                        bufb.at[:, pl.ds(0, vw)], semb).wait()
                    accumulate(bufb, r, ng)

                return 0

            lax.fori_loop(0, DT, rbody, 0)
            cpd.wait()
            accumulate(bufd, DT, ng)
            pltpu.sync_copy(acc1.at[pl.ds(0, vw)], p1_hbm.at[pl.ds(lv0, vw)])
            pltpu.sync_copy(acc2.at[pl.ds(0, vw)], p2_hbm.at[pl.ds(lv0, vw)])

        base = wid * (TPW * 128)

        def chunk_body(ci, _):
            do_chunk(base + ci * CHW, CHW)
            return 0

        lax.fori_loop(0, NFULL, chunk_body, 0)
        if REMW:
            do_chunk(base + NFULL * CHW, REMW)

        @pl.when(wid == NW - 1)
        def _():
            def tail_chunk(tc, _):
                pltpu.sync_copy(tail2_hbm.at[:, pl.ds(tc * 128, 128)], tbuf)

                def tg_body(g, _):
                    sl = pl.ds(g * L, L)
                    a1 = jnp.zeros((L,), jnp.float32)
                    a2 = jnp.zeros((L,), jnp.float32)
                    for db in range(0, 304, L):
                        wv1 = w1_v[pl.ds(db, L)]
                        wv2 = w2_v[pl.ds(db, L)]
                        for j in range(L):
                            v = tbuf[db + j, sl]
                            a1 = a1 + v * wv1[j]
                            a2 = a2 + v * wv2[j]
                    acc1[sl] = a1
                    acc2[sl] = a2
                    return 0

                lax.fori_loop(0, 8, tg_body, 0)
                lv0 = TAIL0 - VS0 + tc * 128
                pltpu.sync_copy(acc1.at[pl.ds(0, 128)],
                                p1_hbm.at[pl.ds(lv0, 128)])
                pltpu.sync_copy(acc2.at[pl.ds(0, 128)],
                                p2_hbm.at[pl.ds(lv0, 128)])
                return 0

            lax.fori_loop(0, NTAILC, tail_chunk, 0)

    return k


def _gather_kernel(B):
    bpw = B // NW

    mesh = plsc.VectorSubcoreMesh(core_axis_name="c", subcore_axis_name="s")

    @functools.partial(
        pl.kernel,
        out_type=jax.ShapeDtypeStruct((B,), jnp.float32),
        mesh=mesh,
        compiler_params=pltpu.CompilerParams(
            needs_layout_passes=False, use_tc_tiling_on_sc=False),
        scratch_types=[
            pltpu.VMEM((2 * bpw,), jnp.int32),
            pltpu.VMEM((bpw,), jnp.int32),
            pltpu.VMEM((bpw,), jnp.int32),
            pltpu.VMEM((bpw, 8), jnp.float32),
            pltpu.VMEM((bpw, 8), jnp.float32),
            pltpu.VMEM((L,), jnp.float32),
            pltpu.VMEM((bpw,), jnp.float32),
            pltpu.SemaphoreType.DMA,
        ],
    )
    def k(idx_hbm, p1_hbm, p2_hbm, b_hbm, out_hbm,
          idx_v, blk1_v, blk2_v, g1_v, g2_v, b_v, out_v, sem):
        wid = lax.axis_index("s") * NC + lax.axis_index("c")
        base = wid * bpw
        pltpu.sync_copy(idx_hbm.at[pl.ds(2 * base, 2 * bpw)], idx_v)
        pltpu.sync_copy(b_hbm, b_v)
        lane = lax.iota(jnp.int32, L)

        def blk_body(g, _):
            e = g * L + lane
            i1 = plsc.load_gather(idx_v, [2 * e])
            i2 = plsc.load_gather(idx_v, [2 * e + 1])
            blk1_v[pl.ds(g * L, L)] = i1 >> 3
            blk2_v[pl.ds(g * L, L)] = i2 >> 3
            return 0

        lax.fori_loop(0, bpw // L, blk_body, 0)
        c1 = pltpu.async_copy(p1_hbm.at[blk1_v], g1_v, sem)
        c2 = pltpu.async_copy(p2_hbm.at[blk2_v], g2_v, sem)
        c1.wait()
        c2.wait()
        bv = b_v[...]

        def out_body(g, _):
            e = g * L + lane
            i1 = plsc.load_gather(idx_v, [2 * e])
            i2 = plsc.load_gather(idx_v, [2 * e + 1])
            v1 = plsc.load_gather(g1_v, [e, i1 & 7])
            v2 = plsc.load_gather(g2_v, [e, i2 & 7])
            out_v[pl.ds(g * L, L)] = jnp.maximum(v1 + v2 + bv, 0.0)
            return 0

        lax.fori_loop(0, bpw // L, out_body, 0)
        pltpu.sync_copy(out_v, out_hbm.at[pl.ds(base, bpw)])

    return k


def kernel(input, table, W, b):
    B = input.shape[0]
    V, D = table.shape
    tableT = table.T
    idx_flat = input.reshape(-1).astype(jnp.int32)
    w_flat = W.reshape(-1).astype(jnp.float32)
    w2t = jnp.stack([w_flat[:D], w_flat[D:]])
    pad = jnp.zeros((WPAD - D,), jnp.float32)
    w1 = jnp.concatenate([w_flat[:D], pad])
    w2 = jnp.concatenate([w_flat[D:], pad])
    b_vec = jnp.full((L,), b[0], dtype=jnp.float32)
    taild = jnp.concatenate(
        [table[VS0:, DFULL:D].T,
         jnp.zeros((8 - (D - DFULL), V - VS0), jnp.float32)], 0)
    taild = jnp.pad(taild, ((0, 0), (0, VSCPAD - (V - VS0))))
    tail2 = jnp.pad(table[TAIL0:].T,
                    ((0, 304 - D), (0, NTAILC * 128 - (V - TAIL0))))
    p1s, p2s = _matvec_sc()(tableT, taild, tail2, w1, w2)
    p1t, p2t = _matvec_tc(D)(w2t, tableT)
    p1 = jnp.concatenate([p1t.reshape(-1), p1s[:V - VS0]])
    p2 = jnp.concatenate([p2t.reshape(-1), p2s[:V - VS0]])
    p1b = p1.reshape(V // 8, 8)
    p2b = p2.reshape(V // 8, 8)
    return _gather_kernel(B)(idx_flat, p1b, p2b, b_vec)


# speedup vs baseline: 1.0040x; 1.0040x over previous
import functools

import jax
import jax.numpy as jnp
from jax import lax
from jax.experimental import pallas as pl
from jax.experimental.pallas import tpu as pltpu
from jax.experimental.pallas import tpu_sc as plsc

NC = 2
NS = 16
NW = NC * NS
L = 16
WPAD = 312

BW = 8192
VS0 = 770048

TPW = 56
CHT = 16
CHW = CHT * 128
NFULL = 3
REMW = (TPW - NFULL * CHT) * 128
TAIL0 = VS0 + NW * TPW * 128
NTAILC = 5
VSCPAD = TAIL0 - VS0 + NTAILC * 128
DFULL = 296


def _matvec_tc(D):
    def mv(w_ref, x_ref, o1_ref, o2_ref):
        p = jax.lax.dot_general(
            w_ref[...], x_ref[...], (((1,), (0,)), ((), ())),
            preferred_element_type=jnp.float32)
        o1_ref[...] = p[0:1]
        o2_ref[...] = p[1:2]

    return pl.pallas_call(
        mv,
        grid=(VS0 // BW,),
        in_specs=[
            pl.BlockSpec((2, D), lambda i: (0, 0)),
            pl.BlockSpec((D, BW), lambda i: (0, i)),
        ],
        out_specs=[
            pl.BlockSpec((1, BW), lambda i: (0, i)),
            pl.BlockSpec((1, BW), lambda i: (0, i)),
        ],
        out_shape=[
            jax.ShapeDtypeStruct((1, VS0), jnp.float32),
            jax.ShapeDtypeStruct((1, VS0), jnp.float32),
        ],
    )


def _matvec_sc():
    DT = DFULL // 8

    mesh = plsc.VectorSubcoreMesh(core_axis_name="c", subcore_axis_name="s")

    @functools.partial(
        pl.kernel,
        out_type=(
            jax.ShapeDtypeStruct((VSCPAD,), jnp.float32),
            jax.ShapeDtypeStruct((VSCPAD,), jnp.float32),
        ),
        mesh=mesh,
        compiler_params=pltpu.CompilerParams(use_tc_tiling_on_sc=True),
        scratch_types=[
            pltpu.VMEM((8, CHW), jnp.float32),
            pltpu.VMEM((8, CHW), jnp.float32),
            pltpu.VMEM((8, CHW), jnp.float32),
            pltpu.VMEM((CHW,), jnp.float32),
            pltpu.VMEM((CHW,), jnp.float32),
            pltpu.VMEM((WPAD,), jnp.float32),
            pltpu.VMEM((WPAD,), jnp.float32),
            pltpu.VMEM((304, 128), jnp.float32),
            pltpu.SemaphoreType.DMA,
            pltpu.SemaphoreType.DMA,
            pltpu.SemaphoreType.DMA,
        ],
    )
    def k(tabT_hbm, taild_hbm, tail2_hbm, w1_hbm, w2_hbm, p1_hbm, p2_hbm,
          bufa, bufb, bufd, acc1, acc2, w1_v, w2_v, tbuf,
          sema, semb, semd):
        wid = lax.axis_index("s") * NC + lax.axis_index("c")
        pltpu.sync_copy(w1_hbm, w1_v)
        pltpu.sync_copy(w2_hbm, w2_v)

        def accumulate(strip, r, ng):
            wv1 = w1_v[pl.ds(8 * r, L)]
            wv2 = w2_v[pl.ds(8 * r, L)]

            def gbody(g, _):
                sl = pl.ds(g * L, L)
                a1 = acc1[sl]
                a2 = acc2[sl]
                for kk in range(8):
                    v = strip[kk, sl]
                    a1 = a1 + v * wv1[kk]
                    a2 = a2 + v * wv2[kk]
                acc1[sl] = a1
                acc2[sl] = a2
                return 0

            lax.fori_loop(0, ng, gbody, 0)

        def do_chunk(lv0, vw):
            ng = vw // L

            def zbody(g, _):
                z = jnp.zeros((L,), jnp.float32)
                acc1[pl.ds(g * L, L)] = z
                acc2[pl.ds(g * L, L)] = z
                return 0

            lax.fori_loop(0, ng, zbody, 0)

            cpd = pltpu.async_copy(
                taild_hbm.at[:, pl.ds(lv0, vw)],
                bufd.at[:, pl.ds(0, vw)], semd)
            pltpu.async_copy(
                tabT_hbm.at[pl.ds(0, 8), pl.ds(VS0 + lv0, vw)],
                bufa.at[:, pl.ds(0, vw)], sema)

            def rbody(r, _):
                @pl.when(r < DT - 1)
                def _():
                    nxt = r + 1

                    @pl.when(nxt % 2 == 0)
                    def _():
                        pltpu.async_copy(
                            tabT_hbm.at[pl.ds(8 * nxt, 8),
                                        pl.ds(VS0 + lv0, vw)],
                            bufa.at[:, pl.ds(0, vw)], sema)

                    @pl.when(nxt % 2 == 1)
                    def _():
                        pltpu.async_copy(
                            tabT_hbm.at[pl.ds(8 * nxt, 8),
                                        pl.ds(VS0 + lv0, vw)],
                            bufb.at[:, pl.ds(0, vw)], semb)

                @pl.when(r % 2 == 0)
                def _():
                    pltpu.make_async_copy(
                        tabT_hbm.at[pl.ds(0, 8), pl.ds(VS0 + lv0, vw)],
                        bufa.at[:, pl.ds(0, vw)], sema).wait()
                    accumulate(bufa, r, ng)

                @pl.when(r % 2 == 1)
                def _():
                    pltpu.make_async_copy(
                        tabT_hbm.at[pl.ds(0, 8), pl.ds(VS0 + lv0, vw)],
                        bufb.at[:, pl.ds(0, vw)], semb).wait()
                    accumulate(bufb, r, ng)

                return 0

            lax.fori_loop(0, DT, rbody, 0)
            cpd.wait()
            accumulate(bufd, DT, ng)
            pltpu.sync_copy(acc1.at[pl.ds(0, vw)], p1_hbm.at[pl.ds(lv0, vw)])
            pltpu.sync_copy(acc2.at[pl.ds(0, vw)], p2_hbm.at[pl.ds(lv0, vw)])

        base = wid * (TPW * 128)

        def chunk_body(ci, _):
            do_chunk(base + ci * CHW, CHW)
            return 0

        lax.fori_loop(0, NFULL, chunk_body, 0)
        if REMW:
            do_chunk(base + NFULL * CHW, REMW)

        @pl.when(wid == NW - 1)
        def _():
            def tail_chunk(tc, _):
                pltpu.sync_copy(tail2_hbm.at[:, pl.ds(tc * 128, 128)], tbuf)

                def tg_body(g, _):
                    sl = pl.ds(g * L, L)
                    a1 = jnp.zeros((L,), jnp.float32)
                    a2 = jnp.zeros((L,), jnp.float32)
                    for db in range(0, 304, L):
                        wv1 = w1_v[pl.ds(db, L)]
                        wv2 = w2_v[pl.ds(db, L)]
                        for j in range(L):
                            v = tbuf[db + j, sl]
                            a1 = a1 + v * wv1[j]
                            a2 = a2 + v * wv2[j]
                    acc1[sl] = a1
                    acc2[sl] = a2
                    return 0

                lax.fori_loop(0, 8, tg_body, 0)
                lv0 = TAIL0 - VS0 + tc * 128
                pltpu.sync_copy(acc1.at[pl.ds(0, 128)],
                                p1_hbm.at[pl.ds(lv0, 128)])
                pltpu.sync_copy(acc2.at[pl.ds(0, 128)],
                                p2_hbm.at[pl.ds(lv0, 128)])
                return 0

            lax.fori_loop(0, NTAILC, tail_chunk, 0)

    return k


def _gather_kernel(B):
    bpw = B // NW

    mesh = plsc.VectorSubcoreMesh(core_axis_name="c", subcore_axis_name="s")

    @functools.partial(
        pl.kernel,
        out_type=jax.ShapeDtypeStruct((B,), jnp.float32),
        mesh=mesh,
        compiler_params=pltpu.CompilerParams(
            needs_layout_passes=False, use_tc_tiling_on_sc=False),
        scratch_types=[
            pltpu.VMEM((2 * bpw,), jnp.int32),
            pltpu.VMEM((bpw,), jnp.int32),
            pltpu.VMEM((bpw,), jnp.int32),
            pltpu.VMEM((bpw, 8), jnp.float32),
            pltpu.VMEM((bpw, 8), jnp.float32),
            pltpu.VMEM((L,), jnp.float32),
            pltpu.VMEM((bpw,), jnp.float32),
            pltpu.SemaphoreType.DMA,
        ],
    )
    def k(idx_hbm, p1_hbm, p2_hbm, b_hbm, out_hbm,
          idx_v, blk1_v, blk2_v, g1_v, g2_v, b_v, out_v, sem):
        wid = lax.axis_index("s") * NC + lax.axis_index("c")
        base = wid * bpw
        pltpu.sync_copy(idx_hbm.at[pl.ds(2 * base, 2 * bpw)], idx_v)
        pltpu.sync_copy(b_hbm, b_v)
        lane = lax.iota(jnp.int32, L)

        def blk_body(g, _):
            e = g * L + lane
            i1 = plsc.load_gather(idx_v, [2 * e])
            i2 = plsc.load_gather(idx_v, [2 * e + 1])
            blk1_v[pl.ds(g * L, L)] = i1 >> 3
            blk2_v[pl.ds(g * L, L)] = i2 >> 3
            return 0

        lax.fori_loop(0, bpw // L, blk_body, 0)
        c1 = pltpu.async_copy(p1_hbm.at[blk1_v], g1_v, sem)
        c2 = pltpu.async_copy(p2_hbm.at[blk2_v], g2_v, sem)
        c1.wait()
        c2.wait()
        bv = b_v[...]

        def out_body(g, _):
            e = g * L + lane
            i1 = plsc.load_gather(idx_v, [2 * e])
            i2 = plsc.load_gather(idx_v, [2 * e + 1])
            v1 = plsc.load_gather(g1_v, [e, i1 & 7])
            v2 = plsc.load_gather(g2_v, [e, i2 & 7])
            out_v[pl.ds(g * L, L)] = jnp.maximum(v1 + v2 + bv, 0.0)
            return 0

        lax.fori_loop(0, bpw // L, out_body, 0)
        pltpu.sync_copy(out_v, out_hbm.at[pl.ds(base, bpw)])

    return k


def kernel(input, table, W, b):
    B = input.shape[0]
    V, D = table.shape
    tableT = table.T
    idx_flat = input.reshape(-1).astype(jnp.int32)
    w_flat = W.reshape(-1).astype(jnp.float32)
    w2t = jnp.stack([w_flat[:D], w_flat[D:]])
    pad = jnp.zeros((WPAD - D,), jnp.float32)
    w1 = jnp.concatenate([w_flat[:D], pad])
    w2 = jnp.concatenate([w_flat[D:], pad])
    b_vec = jnp.full((L,), b[0], dtype=jnp.float32)
    taild = jnp.concatenate(
        [table[VS0:, DFULL:D].T,
         jnp.zeros((8 - (D - DFULL), V - VS0), jnp.float32)], 0)
    taild = jnp.pad(taild, ((0, 0), (0, VSCPAD - (V - VS0))))
    tail2 = jnp.pad(table[TAIL0:].T,
                    ((0, 304 - D), (0, NTAILC * 128 - (V - TAIL0))))
    p1s, p2s = _matvec_sc()(tableT, taild, tail2, w1, w2)
    p1t, p2t = _matvec_tc(D)(w2t, tableT)
    p1 = jnp.concatenate([p1t.reshape(-1), p1s[:V - VS0]])
    p2 = jnp.concatenate([p2t.reshape(-1), p2s[:V - VS0]])
    p1b = p1.reshape(V // 8, 8)
    p2b = p2.reshape(V // 8, 8)
    return _gather_kernel(B)(idx_flat, p1b, p2b, b_vec)


# speedup vs baseline: 1.0088x; 1.0048x over previous
import functools

import jax
import jax.numpy as jnp
from jax import lax
from jax.experimental import pallas as pl
from jax.experimental.pallas import tpu as pltpu
from jax.experimental.pallas import tpu_sc as plsc

NC = 2
NS = 16
NW = NC * NS
L = 16
WPAD = 312

BW = 8192
VS0 = 802816

TPW = 48
CHT = 16
CHW = CHT * 128
NFULL = 3
REMW = (TPW - NFULL * CHT) * 128
TAIL0 = VS0 + NW * TPW * 128
NTAILC = 5
VSCPAD = TAIL0 - VS0 + NTAILC * 128
DFULL = 296


def _matvec_tc(D):
    def mv(w_ref, x_ref, o1_ref, o2_ref):
        p = jax.lax.dot_general(
            w_ref[...], x_ref[...], (((1,), (0,)), ((), ())),
            preferred_element_type=jnp.float32)
        o1_ref[...] = p[0:1]
        o2_ref[...] = p[1:2]

    return pl.pallas_call(
        mv,
        grid=(VS0 // BW,),
        in_specs=[
            pl.BlockSpec((2, D), lambda i: (0, 0)),
            pl.BlockSpec((D, BW), lambda i: (0, i)),
        ],
        out_specs=[
            pl.BlockSpec((1, BW), lambda i: (0, i)),
            pl.BlockSpec((1, BW), lambda i: (0, i)),
        ],
        out_shape=[
            jax.ShapeDtypeStruct((1, VS0), jnp.float32),
            jax.ShapeDtypeStruct((1, VS0), jnp.float32),
        ],
    )


def _matvec_sc():
    DT = DFULL // 8

    mesh = plsc.VectorSubcoreMesh(core_axis_name="c", subcore_axis_name="s")

    @functools.partial(
        pl.kernel,
        out_type=(
            jax.ShapeDtypeStruct((VSCPAD,), jnp.float32),
            jax.ShapeDtypeStruct((VSCPAD,), jnp.float32),
        ),
        mesh=mesh,
        compiler_params=pltpu.CompilerParams(use_tc_tiling_on_sc=True),
        scratch_types=[
            pltpu.VMEM((8, CHW), jnp.float32),
            pltpu.VMEM((8, CHW), jnp.float32),
            pltpu.VMEM((8, CHW), jnp.float32),
            pltpu.VMEM((CHW,), jnp.float32),
            pltpu.VMEM((CHW,), jnp.float32),
            pltpu.VMEM((WPAD,), jnp.float32),
            pltpu.VMEM((WPAD,), jnp.float32),
            pltpu.VMEM((304, 128), jnp.float32),
            pltpu.SemaphoreType.DMA,
            pltpu.SemaphoreType.DMA,
            pltpu.SemaphoreType.DMA,
        ],
    )
    def k(tabT_hbm, taild_hbm, tail2_hbm, w1_hbm, w2_hbm, p1_hbm, p2_hbm,
          bufa, bufb, bufd, acc1, acc2, w1_v, w2_v, tbuf,
          sema, semb, semd):
        wid = lax.axis_index("s") * NC + lax.axis_index("c")
        pltpu.sync_copy(w1_hbm, w1_v)
        pltpu.sync_copy(w2_hbm, w2_v)

        def accumulate(strip, r, ng):
            wv1 = w1_v[pl.ds(8 * r, L)]
            wv2 = w2_v[pl.ds(8 * r, L)]

            def gbody(g, _):
                sl = pl.ds(g * L, L)
                a1 = acc1[sl]
                a2 = acc2[sl]
                for kk in range(8):
                    v = strip[kk, sl]
                    a1 = a1 + v * wv1[kk]
                    a2 = a2 + v * wv2[kk]
                acc1[sl] = a1
                acc2[sl] = a2
                return 0

            lax.fori_loop(0, ng, gbody, 0)

        def do_chunk(lv0, vw):
            ng = vw // L

            def zbody(g, _):
                z = jnp.zeros((L,), jnp.float32)
                acc1[pl.ds(g * L, L)] = z
                acc2[pl.ds(g * L, L)] = z
                return 0

            lax.fori_loop(0, ng, zbody, 0)

            cpd = pltpu.async_copy(
                taild_hbm.at[:, pl.ds(lv0, vw)],
                bufd.at[:, pl.ds(0, vw)], semd)
            pltpu.async_copy(
                tabT_hbm.at[pl.ds(0, 8), pl.ds(VS0 + lv0, vw)],
                bufa.at[:, pl.ds(0, vw)], sema)

            def rbody(r, _):
                @pl.when(r < DT - 1)
                def _():
                    nxt = r + 1

                    @pl.when(nxt % 2 == 0)
                    def _():
                        pltpu.async_copy(
                            tabT_hbm.at[pl.ds(8 * nxt, 8),
                                        pl.ds(VS0 + lv0, vw)],
                            bufa.at[:, pl.ds(0, vw)], sema)

                    @pl.when(nxt % 2 == 1)
                    def _():
                        pltpu.async_copy(
                            tabT_hbm.at[pl.ds(8 * nxt, 8),
                                        pl.ds(VS0 + lv0, vw)],
                            bufb.at[:, pl.ds(0, vw)], semb)

                @pl.when(r % 2 == 0)
                def _():
                    pltpu.make_async_copy(
                        tabT_hbm.at[pl.ds(0, 8), pl.ds(VS0 + lv0, vw)],
                        bufa.at[:, pl.ds(0, vw)], sema).wait()
                    accumulate(bufa, r, ng)

                @pl.when(r % 2 == 1)
                def _():
                    pltpu.make_async_copy(
                        tabT_hbm.at[pl.ds(0, 8), pl.ds(VS0 + lv0, vw)],
                        bufb.at[:, pl.ds(0, vw)], semb).wait()
                    accumulate(bufb, r, ng)

                return 0

            lax.fori_loop(0, DT, rbody, 0)
            cpd.wait()
            accumulate(bufd, DT, ng)
            pltpu.sync_copy(acc1.at[pl.ds(0, vw)], p1_hbm.at[pl.ds(lv0, vw)])
            pltpu.sync_copy(acc2.at[pl.ds(0, vw)], p2_hbm.at[pl.ds(lv0, vw)])

        base = wid * (TPW * 128)

        def chunk_body(ci, _):
            do_chunk(base + ci * CHW, CHW)
            return 0

        lax.fori_loop(0, NFULL, chunk_body, 0)
        if REMW:
            do_chunk(base + NFULL * CHW, REMW)

        @pl.when(wid == NW - 1)
        def _():
            def tail_chunk(tc, _):
                pltpu.sync_copy(tail2_hbm.at[:, pl.ds(tc * 128, 128)], tbuf)

                def tg_body(g, _):
                    sl = pl.ds(g * L, L)
                    a1 = jnp.zeros((L,), jnp.float32)
                    a2 = jnp.zeros((L,), jnp.float32)
                    for db in range(0, 304, L):
                        wv1 = w1_v[pl.ds(db, L)]
                        wv2 = w2_v[pl.ds(db, L)]
                        for j in range(L):
                            v = tbuf[db + j, sl]
                            a1 = a1 + v * wv1[j]
                            a2 = a2 + v * wv2[j]
                    acc1[sl] = a1
                    acc2[sl] = a2
                    return 0

                lax.fori_loop(0, 8, tg_body, 0)
                lv0 = TAIL0 - VS0 + tc * 128
                pltpu.sync_copy(acc1.at[pl.ds(0, 128)],
                                p1_hbm.at[pl.ds(lv0, 128)])
                pltpu.sync_copy(acc2.at[pl.ds(0, 128)],
                                p2_hbm.at[pl.ds(lv0, 128)])
                return 0

            lax.fori_loop(0, NTAILC, tail_chunk, 0)

    return k


def _gather_kernel(B):
    bpw = B // NW

    mesh = plsc.VectorSubcoreMesh(core_axis_name="c", subcore_axis_name="s")

    @functools.partial(
        pl.kernel,
        out_type=jax.ShapeDtypeStruct((B,), jnp.float32),
        mesh=mesh,
        compiler_params=pltpu.CompilerParams(
            needs_layout_passes=False, use_tc_tiling_on_sc=False),
        scratch_types=[
            pltpu.VMEM((2 * bpw,), jnp.int32),
            pltpu.VMEM((bpw,), jnp.int32),
            pltpu.VMEM((bpw,), jnp.int32),
            pltpu.VMEM((bpw, 8), jnp.float32),
            pltpu.VMEM((bpw, 8), jnp.float32),
            pltpu.VMEM((L,), jnp.float32),
            pltpu.VMEM((bpw,), jnp.float32),
            pltpu.SemaphoreType.DMA,
        ],
    )
    def k(idx_hbm, p1_hbm, p2_hbm, b_hbm, out_hbm,
          idx_v, blk1_v, blk2_v, g1_v, g2_v, b_v, out_v, sem):
        wid = lax.axis_index("s") * NC + lax.axis_index("c")
        base = wid * bpw
        pltpu.sync_copy(idx_hbm.at[pl.ds(2 * base, 2 * bpw)], idx_v)
        pltpu.sync_copy(b_hbm, b_v)
        lane = lax.iota(jnp.int32, L)

        def blk_body(g, _):
            e = g * L + lane
            i1 = plsc.load_gather(idx_v, [2 * e])
            i2 = plsc.load_gather(idx_v, [2 * e + 1])
            blk1_v[pl.ds(g * L, L)] = i1 >> 3
            blk2_v[pl.ds(g * L, L)] = i2 >> 3
            return 0

        lax.fori_loop(0, bpw // L, blk_body, 0)
        c1 = pltpu.async_copy(p1_hbm.at[blk1_v], g1_v, sem)
        c2 = pltpu.async_copy(p2_hbm.at[blk2_v], g2_v, sem)
        c1.wait()
        c2.wait()
        bv = b_v[...]

        def out_body(g, _):
            e = g * L + lane
            i1 = plsc.load_gather(idx_v, [2 * e])
            i2 = plsc.load_gather(idx_v, [2 * e + 1])
            v1 = plsc.load_gather(g1_v, [e, i1 & 7])
            v2 = plsc.load_gather(g2_v, [e, i2 & 7])
            out_v[pl.ds(g * L, L)] = jnp.maximum(v1 + v2 + bv, 0.0)
            return 0

        lax.fori_loop(0, bpw // L, out_body, 0)
        pltpu.sync_copy(out_v, out_hbm.at[pl.ds(base, bpw)])

    return k


def kernel(input, table, W, b):
    B = input.shape[0]
    V, D = table.shape
    tableT = table.T
    idx_flat = input.reshape(-1).astype(jnp.int32)
    w_flat = W.reshape(-1).astype(jnp.float32)
    w2t = jnp.stack([w_flat[:D], w_flat[D:]])
    pad = jnp.zeros((WPAD - D,), jnp.float32)
    w1 = jnp.concatenate([w_flat[:D], pad])
    w2 = jnp.concatenate([w_flat[D:], pad])
    b_vec = jnp.full((L,), b[0], dtype=jnp.float32)
    taild = jnp.concatenate(
        [table[VS0:, DFULL:D].T,
         jnp.zeros((8 - (D - DFULL), V - VS0), jnp.float32)], 0)
    taild = jnp.pad(taild, ((0, 0), (0, VSCPAD - (V - VS0))))
    tail2 = jnp.pad(table[TAIL0:].T,
                    ((0, 304 - D), (0, NTAILC * 128 - (V - TAIL0))))
    p1s, p2s = _matvec_sc()(tableT, taild, tail2, w1, w2)
    p1t, p2t = _matvec_tc(D)(w2t, tableT)
    p1 = jnp.concatenate([p1t.reshape(-1), p1s[:V - VS0]])
    p2 = jnp.concatenate([p2t.reshape(-1), p2s[:V - VS0]])
    p1b = p1.reshape(V // 8, 8)
    p2b = p2.reshape(V // 8, 8)
    return _gather_kernel(B)(idx_flat, p1b, p2b, b_vec)


# speedup vs baseline: 1.0137x; 1.0048x over previous
import functools

import jax
import jax.numpy as jnp
from jax import lax
from jax.experimental import pallas as pl
from jax.experimental.pallas import tpu as pltpu
from jax.experimental.pallas import tpu_sc as plsc

NC = 2
NS = 16
NW = NC * NS
L = 16
WPAD = 312

BW = 8192
VS0 = 835584

TPW = 40
CHT = 16
CHW = CHT * 128
NFULL = 2
REMW = (TPW - NFULL * CHT) * 128
TAIL0 = VS0 + NW * TPW * 128
NTAILC = 5
VSCPAD = TAIL0 - VS0 + NTAILC * 128
DFULL = 296


def _matvec_tc(D):
    def mv(w_ref, x_ref, o1_ref, o2_ref):
        p = jax.lax.dot_general(
            w_ref[...], x_ref[...], (((1,), (0,)), ((), ())),
            preferred_element_type=jnp.float32)
        o1_ref[...] = p[0:1]
        o2_ref[...] = p[1:2]

    return pl.pallas_call(
        mv,
        grid=(VS0 // BW,),
        in_specs=[
            pl.BlockSpec((2, D), lambda i: (0, 0)),
            pl.BlockSpec((D, BW), lambda i: (0, i)),
        ],
        out_specs=[
            pl.BlockSpec((1, BW), lambda i: (0, i)),
            pl.BlockSpec((1, BW), lambda i: (0, i)),
        ],
        out_shape=[
            jax.ShapeDtypeStruct((1, VS0), jnp.float32),
            jax.ShapeDtypeStruct((1, VS0), jnp.float32),
        ],
    )


def _matvec_sc():
    DT = DFULL // 8

    mesh = plsc.VectorSubcoreMesh(core_axis_name="c", subcore_axis_name="s")

    @functools.partial(
        pl.kernel,
        out_type=(
            jax.ShapeDtypeStruct((VSCPAD,), jnp.float32),
            jax.ShapeDtypeStruct((VSCPAD,), jnp.float32),
        ),
        mesh=mesh,
        compiler_params=pltpu.CompilerParams(use_tc_tiling_on_sc=True),
        scratch_types=[
            pltpu.VMEM((8, CHW), jnp.float32),
            pltpu.VMEM((8, CHW), jnp.float32),
            pltpu.VMEM((8, CHW), jnp.float32),
            pltpu.VMEM((CHW,), jnp.float32),
            pltpu.VMEM((CHW,), jnp.float32),
            pltpu.VMEM((WPAD,), jnp.float32),
            pltpu.VMEM((WPAD,), jnp.float32),
            pltpu.VMEM((304, 128), jnp.float32),
            pltpu.SemaphoreType.DMA,
            pltpu.SemaphoreType.DMA,
            pltpu.SemaphoreType.DMA,
        ],
    )
    def k(tabT_hbm, taild_hbm, tail2_hbm, w1_hbm, w2_hbm, p1_hbm, p2_hbm,
          bufa, bufb, bufd, acc1, acc2, w1_v, w2_v, tbuf,
          sema, semb, semd):
        wid = lax.axis_index("s") * NC + lax.axis_index("c")
        pltpu.sync_copy(w1_hbm, w1_v)
        pltpu.sync_copy(w2_hbm, w2_v)

        def accumulate(strip, r, ng):
            wv1 = w1_v[pl.ds(8 * r, L)]
            wv2 = w2_v[pl.ds(8 * r, L)]

            def gbody(g, _):
                sl = pl.ds(g * L, L)
                a1 = acc1[sl]
                a2 = acc2[sl]
                for kk in range(8):
                    v = strip[kk, sl]
                    a1 = a1 + v * wv1[kk]
                    a2 = a2 + v * wv2[kk]
                acc1[sl] = a1
                acc2[sl] = a2
                return 0

            lax.fori_loop(0, ng, gbody, 0)

        def do_chunk(lv0, vw):
            ng = vw // L

            def zbody(g, _):
                z = jnp.zeros((L,), jnp.float32)
                acc1[pl.ds(g * L, L)] = z
                acc2[pl.ds(g * L, L)] = z
                return 0

            lax.fori_loop(0, ng, zbody, 0)

            cpd = pltpu.async_copy(
                taild_hbm.at[:, pl.ds(lv0, vw)],
                bufd.at[:, pl.ds(0, vw)], semd)
            pltpu.async_copy(
                tabT_hbm.at[pl.ds(0, 8), pl.ds(VS0 + lv0, vw)],
                bufa.at[:, pl.ds(0, vw)], sema)

            def rbody(r, _):
                @pl.when(r < DT - 1)
                def _():
                    nxt = r + 1

                    @pl.when(nxt % 2 == 0)
                    def _():
                        pltpu.async_copy(
                            tabT_hbm.at[pl.ds(8 * nxt, 8),
                                        pl.ds(VS0 + lv0, vw)],
                            bufa.at[:, pl.ds(0, vw)], sema)

                    @pl.when(nxt % 2 == 1)
                    def _():
                        pltpu.async_copy(
                            tabT_hbm.at[pl.ds(8 * nxt, 8),
                                        pl.ds(VS0 + lv0, vw)],
                            bufb.at[:, pl.ds(0, vw)], semb)

                @pl.when(r % 2 == 0)
                def _():
                    pltpu.make_async_copy(
                        tabT_hbm.at[pl.ds(0, 8), pl.ds(VS0 + lv0, vw)],
                        bufa.at[:, pl.ds(0, vw)], sema).wait()
                    accumulate(bufa, r, ng)

                @pl.when(r % 2 == 1)
                def _():
                    pltpu.make_async_copy(
                        tabT_hbm.at[pl.ds(0, 8), pl.ds(VS0 + lv0, vw)],
                        bufb.at[:, pl.ds(0, vw)], semb).wait()
                    accumulate(bufb, r, ng)

                return 0

            lax.fori_loop(0, DT, rbody, 0)
            cpd.wait()
            accumulate(bufd, DT, ng)
            pltpu.sync_copy(acc1.at[pl.ds(0, vw)], p1_hbm.at[pl.ds(lv0, vw)])
            pltpu.sync_copy(acc2.at[pl.ds(0, vw)], p2_hbm.at[pl.ds(lv0, vw)])

        base = wid * (TPW * 128)

        def chunk_body(ci, _):
            do_chunk(base + ci * CHW, CHW)
            return 0

        lax.fori_loop(0, NFULL, chunk_body, 0)
        if REMW:
            do_chunk(base + NFULL * CHW, REMW)

        @pl.when(wid == NW - 1)
        def _():
            def tail_chunk(tc, _):
                pltpu.sync_copy(tail2_hbm.at[:, pl.ds(tc * 128, 128)], tbuf)

                def tg_body(g, _):
                    sl = pl.ds(g * L, L)
                    a1 = jnp.zeros((L,), jnp.float32)
                    a2 = jnp.zeros((L,), jnp.float32)
                    for db in range(0, 304, L):
                        wv1 = w1_v[pl.ds(db, L)]
                        wv2 = w2_v[pl.ds(db, L)]
                        for j in range(L):
                            v = tbuf[db + j, sl]
                            a1 = a1 + v * wv1[j]
                            a2 = a2 + v * wv2[j]
                    acc1[sl] = a1
                    acc2[sl] = a2
                    return 0

                lax.fori_loop(0, 8, tg_body, 0)
                lv0 = TAIL0 - VS0 + tc * 128
                pltpu.sync_copy(acc1.at[pl.ds(0, 128)],
                                p1_hbm.at[pl.ds(lv0, 128)])
                pltpu.sync_copy(acc2.at[pl.ds(0, 128)],
                                p2_hbm.at[pl.ds(lv0, 128)])
                return 0

            lax.fori_loop(0, NTAILC, tail_chunk, 0)

    return k


def _gather_kernel(B):
    bpw = B // NW

    mesh = plsc.VectorSubcoreMesh(core_axis_name="c", subcore_axis_name="s")

    @functools.partial(
        pl.kernel,
        out_type=jax.ShapeDtypeStruct((B,), jnp.float32),
        mesh=mesh,
        compiler_params=pltpu.CompilerParams(
            needs_layout_passes=False, use_tc_tiling_on_sc=False),
        scratch_types=[
            pltpu.VMEM((2 * bpw,), jnp.int32),
            pltpu.VMEM((bpw,), jnp.int32),
            pltpu.VMEM((bpw,), jnp.int32),
            pltpu.VMEM((bpw, 8), jnp.float32),
            pltpu.VMEM((bpw, 8), jnp.float32),
            pltpu.VMEM((L,), jnp.float32),
            pltpu.VMEM((bpw,), jnp.float32),
            pltpu.SemaphoreType.DMA,
        ],
    )
    def k(idx_hbm, p1_hbm, p2_hbm, b_hbm, out_hbm,
          idx_v, blk1_v, blk2_v, g1_v, g2_v, b_v, out_v, sem):
        wid = lax.axis_index("s") * NC + lax.axis_index("c")
        base = wid * bpw
        pltpu.sync_copy(idx_hbm.at[pl.ds(2 * base, 2 * bpw)], idx_v)
        pltpu.sync_copy(b_hbm, b_v)
        lane = lax.iota(jnp.int32, L)

        def blk_body(g, _):
            e = g * L + lane
            i1 = plsc.load_gather(idx_v, [2 * e])
            i2 = plsc.load_gather(idx_v, [2 * e + 1])
            blk1_v[pl.ds(g * L, L)] = i1 >> 3
            blk2_v[pl.ds(g * L, L)] = i2 >> 3
            return 0

        lax.fori_loop(0, bpw // L, blk_body, 0)
        c1 = pltpu.async_copy(p1_hbm.at[blk1_v], g1_v, sem)
        c2 = pltpu.async_copy(p2_hbm.at[blk2_v], g2_v, sem)
        c1.wait()
        c2.wait()
        bv = b_v[...]

        def out_body(g, _):
            e = g * L + lane
            i1 = plsc.load_gather(idx_v, [2 * e])
            i2 = plsc.load_gather(idx_v, [2 * e + 1])
            v1 = plsc.load_gather(g1_v, [e, i1 & 7])
            v2 = plsc.load_gather(g2_v, [e, i2 & 7])
            out_v[pl.ds(g * L, L)] = jnp.maximum(v1 + v2 + bv, 0.0)
            return 0

        lax.fori_loop(0, bpw // L, out_body, 0)
        pltpu.sync_copy(out_v, out_hbm.at[pl.ds(base, bpw)])

    return k


def kernel(input, table, W, b):
    B = input.shape[0]
    V, D = table.shape
    tableT = table.T
    idx_flat = input.reshape(-1).astype(jnp.int32)
    w_flat = W.reshape(-1).astype(jnp.float32)
    w2t = jnp.stack([w_flat[:D], w_flat[D:]])
    pad = jnp.zeros((WPAD - D,), jnp.float32)
    w1 = jnp.concatenate([w_flat[:D], pad])
    w2 = jnp.concatenate([w_flat[D:], pad])
    b_vec = jnp.full((L,), b[0], dtype=jnp.float32)
    taild = jnp.concatenate(
        [table[VS0:, DFULL:D].T,
         jnp.zeros((8 - (D - DFULL), V - VS0), jnp.float32)], 0)
    taild = jnp.pad(taild, ((0, 0), (0, VSCPAD - (V - VS0))))
    tail2 = jnp.pad(table[TAIL0:].T,
                    ((0, 304 - D), (0, NTAILC * 128 - (V - TAIL0))))
    p1s, p2s = _matvec_sc()(tableT, taild, tail2, w1, w2)
    p1t, p2t = _matvec_tc(D)(w2t, tableT)
    p1 = jnp.concatenate([p1t.reshape(-1), p1s[:V - VS0]])
    p2 = jnp.concatenate([p2t.reshape(-1), p2s[:V - VS0]])
    p1b = p1.reshape(V // 8, 8)
    p2b = p2.reshape(V // 8, 8)
    return _gather_kernel(B)(idx_flat, p1b, p2b, b_vec)


# speedup vs baseline: 1.0173x; 1.0035x over previous
import functools

import jax
import jax.numpy as jnp
from jax import lax
from jax.experimental import pallas as pl
from jax.experimental.pallas import tpu as pltpu
from jax.experimental.pallas import tpu_sc as plsc

NC = 2
NS = 16
NW = NC * NS
L = 16
WPAD = 312

BW = 8192
VS0 = 868352

TPW = 32
CHT = 16
CHW = CHT * 128
NFULL = 2
REMW = (TPW - NFULL * CHT) * 128
TAIL0 = VS0 + NW * TPW * 128
NTAILC = 5
VSCPAD = TAIL0 - VS0 + NTAILC * 128
DFULL = 296


def _matvec_tc(D):
    def mv(w_ref, x_ref, o1_ref, o2_ref):
        p = jax.lax.dot_general(
            w_ref[...], x_ref[...], (((1,), (0,)), ((), ())),
            preferred_element_type=jnp.float32)
        o1_ref[...] = p[0:1]
        o2_ref[...] = p[1:2]

    return pl.pallas_call(
        mv,
        grid=(VS0 // BW,),
        in_specs=[
            pl.BlockSpec((2, D), lambda i: (0, 0)),
            pl.BlockSpec((D, BW), lambda i: (0, i)),
        ],
        out_specs=[
            pl.BlockSpec((1, BW), lambda i: (0, i)),
            pl.BlockSpec((1, BW), lambda i: (0, i)),
        ],
        out_shape=[
            jax.ShapeDtypeStruct((1, VS0), jnp.float32),
            jax.ShapeDtypeStruct((1, VS0), jnp.float32),
        ],
    )


def _matvec_sc():
    DT = DFULL // 8

    mesh = plsc.VectorSubcoreMesh(core_axis_name="c", subcore_axis_name="s")

    @functools.partial(
        pl.kernel,
        out_type=(
            jax.ShapeDtypeStruct((VSCPAD,), jnp.float32),
            jax.ShapeDtypeStruct((VSCPAD,), jnp.float32),
        ),
        mesh=mesh,
        compiler_params=pltpu.CompilerParams(use_tc_tiling_on_sc=True),
        scratch_types=[
            pltpu.VMEM((8, CHW), jnp.float32),
            pltpu.VMEM((8, CHW), jnp.float32),
            pltpu.VMEM((8, CHW), jnp.float32),
            pltpu.VMEM((CHW,), jnp.float32),
            pltpu.VMEM((CHW,), jnp.float32),
            pltpu.VMEM((WPAD,), jnp.float32),
            pltpu.VMEM((WPAD,), jnp.float32),
            pltpu.VMEM((304, 128), jnp.float32),
            pltpu.SemaphoreType.DMA,
            pltpu.SemaphoreType.DMA,
            pltpu.SemaphoreType.DMA,
        ],
    )
    def k(tabT_hbm, taild_hbm, tail2_hbm, w1_hbm, w2_hbm, p1_hbm, p2_hbm,
          bufa, bufb, bufd, acc1, acc2, w1_v, w2_v, tbuf,
          sema, semb, semd):
        wid = lax.axis_index("s") * NC + lax.axis_index("c")
        pltpu.sync_copy(w1_hbm, w1_v)
        pltpu.sync_copy(w2_hbm, w2_v)

        def accumulate(strip, r, ng):
            wv1 = w1_v[pl.ds(8 * r, L)]
            wv2 = w2_v[pl.ds(8 * r, L)]

            def gbody(g, _):
                sl = pl.ds(g * L, L)
                a1 = acc1[sl]
                a2 = acc2[sl]
                for kk in range(8):
                    v = strip[kk, sl]
                    a1 = a1 + v * wv1[kk]
                    a2 = a2 + v * wv2[kk]
                acc1[sl] = a1
                acc2[sl] = a2
                return 0

            lax.fori_loop(0, ng, gbody, 0)

        def do_chunk(lv0, vw):
            ng = vw // L

            def zbody(g, _):
                z = jnp.zeros((L,), jnp.float32)
                acc1[pl.ds(g * L, L)] = z
                acc2[pl.ds(g * L, L)] = z
                return 0

            lax.fori_loop(0, ng, zbody, 0)

            cpd = pltpu.async_copy(
                taild_hbm.at[:, pl.ds(lv0, vw)],
                bufd.at[:, pl.ds(0, vw)], semd)
            pltpu.async_copy(
                tabT_hbm.at[pl.ds(0, 8), pl.ds(VS0 + lv0, vw)],
                bufa.at[:, pl.ds(0, vw)], sema)

            def rbody(r, _):
                @pl.when(r < DT - 1)
                def _():
                    nxt = r + 1

                    @pl.when(nxt % 2 == 0)
                    def _():
                        pltpu.async_copy(
                            tabT_hbm.at[pl.ds(8 * nxt, 8),
                                        pl.ds(VS0 + lv0, vw)],
                            bufa.at[:, pl.ds(0, vw)], sema)

                    @pl.when(nxt % 2 == 1)
                    def _():
                        pltpu.async_copy(
                            tabT_hbm.at[pl.ds(8 * nxt, 8),
                                        pl.ds(VS0 + lv0, vw)],
                            bufb.at[:, pl.ds(0, vw)], semb)

                @pl.when(r % 2 == 0)
                def _():
                    pltpu.make_async_copy(
                        tabT_hbm.at[pl.ds(0, 8), pl.ds(VS0 + lv0, vw)],
                        bufa.at[:, pl.ds(0, vw)], sema).wait()
                    accumulate(bufa, r, ng)

                @pl.when(r % 2 == 1)
                def _():
                    pltpu.make_async_copy(
                        tabT_hbm.at[pl.ds(0, 8), pl.ds(VS0 + lv0, vw)],
                        bufb.at[:, pl.ds(0, vw)], semb).wait()
                    accumulate(bufb, r, ng)

                return 0

            lax.fori_loop(0, DT, rbody, 0)
            cpd.wait()
            accumulate(bufd, DT, ng)
            pltpu.sync_copy(acc1.at[pl.ds(0, vw)], p1_hbm.at[pl.ds(lv0, vw)])
            pltpu.sync_copy(acc2.at[pl.ds(0, vw)], p2_hbm.at[pl.ds(lv0, vw)])

        base = wid * (TPW * 128)

        def chunk_body(ci, _):
            do_chunk(base + ci * CHW, CHW)
            return 0

        lax.fori_loop(0, NFULL, chunk_body, 0)
        if REMW:
            do_chunk(base + NFULL * CHW, REMW)

        @pl.when(wid == NW - 1)
        def _():
            def tail_chunk(tc, _):
                pltpu.sync_copy(tail2_hbm.at[:, pl.ds(tc * 128, 128)], tbuf)

                def tg_body(g, _):
                    sl = pl.ds(g * L, L)
                    a1 = jnp.zeros((L,), jnp.float32)
                    a2 = jnp.zeros((L,), jnp.float32)
                    for db in range(0, 304, L):
                        wv1 = w1_v[pl.ds(db, L)]
                        wv2 = w2_v[pl.ds(db, L)]
                        for j in range(L):
                            v = tbuf[db + j, sl]
                            a1 = a1 + v * wv1[j]
                            a2 = a2 + v * wv2[j]
                    acc1[sl] = a1
                    acc2[sl] = a2
                    return 0

                lax.fori_loop(0, 8, tg_body, 0)
                lv0 = TAIL0 - VS0 + tc * 128
                pltpu.sync_copy(acc1.at[pl.ds(0, 128)],
                                p1_hbm.at[pl.ds(lv0, 128)])
                pltpu.sync_copy(acc2.at[pl.ds(0, 128)],
                                p2_hbm.at[pl.ds(lv0, 128)])
                return 0

            lax.fori_loop(0, NTAILC, tail_chunk, 0)

    return k


def _gather_kernel(B):
    bpw = B // NW

    mesh = plsc.VectorSubcoreMesh(core_axis_name="c", subcore_axis_name="s")

    @functools.partial(
        pl.kernel,
        out_type=jax.ShapeDtypeStruct((B,), jnp.float32),
        mesh=mesh,
        compiler_params=pltpu.CompilerParams(
            needs_layout_passes=False, use_tc_tiling_on_sc=False),
        scratch_types=[
            pltpu.VMEM((2 * bpw,), jnp.int32),
            pltpu.VMEM((bpw,), jnp.int32),
            pltpu.VMEM((bpw,), jnp.int32),
            pltpu.VMEM((bpw, 8), jnp.float32),
            pltpu.VMEM((bpw, 8), jnp.float32),
            pltpu.VMEM((L,), jnp.float32),
            pltpu.VMEM((bpw,), jnp.float32),
            pltpu.SemaphoreType.DMA,
        ],
    )
    def k(idx_hbm, p1_hbm, p2_hbm, b_hbm, out_hbm,
          idx_v, blk1_v, blk2_v, g1_v, g2_v, b_v, out_v, sem):
        wid = lax.axis_index("s") * NC + lax.axis_index("c")
        base = wid * bpw
        pltpu.sync_copy(idx_hbm.at[pl.ds(2 * base, 2 * bpw)], idx_v)
        pltpu.sync_copy(b_hbm, b_v)
        lane = lax.iota(jnp.int32, L)

        def blk_body(g, _):
            e = g * L + lane
            i1 = plsc.load_gather(idx_v, [2 * e])
            i2 = plsc.load_gather(idx_v, [2 * e + 1])
            blk1_v[pl.ds(g * L, L)] = i1 >> 3
            blk2_v[pl.ds(g * L, L)] = i2 >> 3
            return 0

        lax.fori_loop(0, bpw // L, blk_body, 0)
        c1 = pltpu.async_copy(p1_hbm.at[blk1_v], g1_v, sem)
        c2 = pltpu.async_copy(p2_hbm.at[blk2_v], g2_v, sem)
        c1.wait()
        c2.wait()
        bv = b_v[...]

        def out_body(g, _):
            e = g * L + lane
            i1 = plsc.load_gather(idx_v, [2 * e])
            i2 = plsc.load_gather(idx_v, [2 * e + 1])
            v1 = plsc.load_gather(g1_v, [e, i1 & 7])
            v2 = plsc.load_gather(g2_v, [e, i2 & 7])
            out_v[pl.ds(g * L, L)] = jnp.maximum(v1 + v2 + bv, 0.0)
            return 0

        lax.fori_loop(0, bpw // L, out_body, 0)
        pltpu.sync_copy(out_v, out_hbm.at[pl.ds(base, bpw)])

    return k


def kernel(input, table, W, b):
    B = input.shape[0]
    V, D = table.shape
    tableT = table.T
    idx_flat = input.reshape(-1).astype(jnp.int32)
    w_flat = W.reshape(-1).astype(jnp.float32)
    w2t = jnp.stack([w_flat[:D], w_flat[D:]])
    pad = jnp.zeros((WPAD - D,), jnp.float32)
    w1 = jnp.concatenate([w_flat[:D], pad])
    w2 = jnp.concatenate([w_flat[D:], pad])
    b_vec = jnp.full((L,), b[0], dtype=jnp.float32)
    taild = jnp.concatenate(
        [table[VS0:, DFULL:D].T,
         jnp.zeros((8 - (D - DFULL), V - VS0), jnp.float32)], 0)
    taild = jnp.pad(taild, ((0, 0), (0, VSCPAD - (V - VS0))))
    tail2 = jnp.pad(table[TAIL0:].T,
                    ((0, 304 - D), (0, NTAILC * 128 - (V - TAIL0))))
    p1s, p2s = _matvec_sc()(tableT, taild, tail2, w1, w2)
    p1t, p2t = _matvec_tc(D)(w2t, tableT)
    p1 = jnp.concatenate([p1t.reshape(-1), p1s[:V - VS0]])
    p2 = jnp.concatenate([p2t.reshape(-1), p2s[:V - VS0]])
    p1b = p1.reshape(V // 8, 8)
    p2b = p2.reshape(V // 8, 8)
    return _gather_kernel(B)(idx_flat, p1b, p2b, b_vec)
